# baseline (speedup 1.0000x reference)
import functools

import jax
import jax.numpy as jnp
from jax import lax
from jax.experimental import pallas as pl
from jax.experimental.pallas import tpu as pltpu
from jax.experimental.pallas import tpu_sc as plsc

TOKENS = 16384
E = 64
K = 8
L = 16
NC = 2
NS = 16
NW = NC * NS
ROWS = TOKENS // NW

_mesh = plsc.VectorSubcoreMesh(core_axis_name="c", subcore_axis_name="s")


@functools.partial(
    pl.kernel,
    out_type=jax.ShapeDtypeStruct((NW, E), jnp.float32),
    mesh=_mesh,
    compiler_params=pltpu.CompilerParams(needs_layout_passes=False),
    scratch_types=[
        pltpu.VMEM((ROWS // 2, E), jnp.float32),
        pltpu.VMEM((ROWS // 2, E), jnp.float32),
        pltpu.VMEM((E,), jnp.float32),
        pltpu.SemaphoreType.DMA,
        pltpu.SemaphoreType.DMA,
    ],
)
def _sc_hist(x_hbm, out_hbm, buf_a, buf_b, acc_v, sem_a, sem_b):
    c = lax.axis_index("c")
    s = lax.axis_index("s")
    wid = s * NC + c
    base = wid * ROWS

    iota = lax.iota(jnp.int32, L)
    zeros = jnp.zeros((L,), jnp.float32)
    ones = jnp.ones((L,), jnp.float32)
    low_half = iota < 8
    shift8 = (iota + 8) & 15
    idx_consts = [iota + L * j for j in range(E // L)]
    for j in range(E // L):
        acc_v[pl.ds(L * j, L)] = zeros

    def merge(ka, va, kb, vb, descending):
        take_a = ka >= kb
        mk = jnp.maximum(ka, kb)
        mv = jnp.where(take_a, va, vb)
        return plsc.sort_key_val(mk, mv, descending=descending)

    def top8(buf, r):
        ks, vs = [], []
        for j in range(E // L):
            k_s, v_s = plsc.sort_key_val(
                buf[r, pl.ds(L * j, L)], idx_consts[j],
                descending=(j % 2 == 0),
            )
            ks.append(k_s)
            vs.append(v_s)
        k01, v01 = merge(ks[0], vs[0], ks[1], vs[1], descending=True)
        k23, v23 = merge(ks[2], vs[2], ks[3], vs[3], descending=False)
        take_a = k01 >= k23
        kv = jnp.maximum(k01, k23)
        vv = jnp.where(take_a, v01, v23)
        kv_s = lax.gather(
            kv, shift8[:, None],
            dimension_numbers=lax.GatherDimensionNumbers(
                offset_dims=(), collapsed_slice_dims=(0,),
                start_index_map=(0,),
            ),
            slice_sizes=(1,),
            mode=lax.GatherScatterMode.PROMISE_IN_BOUNDS,
        )
        sel = jnp.where(low_half, kv >= kv_s, kv > kv_s)
        return vv, sel

    def loop_hist(buf):
        @plsc.parallel_loop(0, ROWS // 2, unroll=4)
        def _(r):
            vv, sel = top8(buf, r)
            plsc.addupdate_scatter(acc_v, [vv], ones, mask=sel)

    CH = ROWS // 2
    h_a = pltpu.async_copy(x_hbm.at[pl.ds(base, CH)], buf_a, sem_a)
    h_b = pltpu.async_copy(x_hbm.at[pl.ds(base + CH, CH)], buf_b, sem_b)
    h_a.wait()
    loop_hist(buf_a)
    h_b.wait()
    loop_hist(buf_b)

    pltpu.sync_copy(acc_v, out_hbm.at[wid])


def _tc_colsum_body(x_ref, o_ref):
    o_ref[...] = jnp.sum(x_ref[...], axis=0, keepdims=True)


def _combine_body(pl_ref, pw_ref, cs_ref, o_ref):
    o_ref[0:1, :] = jnp.sum(pl_ref[...], axis=0, keepdims=True)
    o_ref[1:2, :] = jnp.sum(pw_ref[...], axis=0, keepdims=True)
    o_ref[2:3, :] = cs_ref[...]


def kernel(layer_idx, router_weights, num_experts_per_tok, router_logits):
    p_l = _sc_hist(router_logits)
    p_w = _sc_hist(router_weights)
    colsum = pl.pallas_call(
        _tc_colsum_body,
        out_shape=jax.ShapeDtypeStruct((1, E), jnp.float32),
    )(router_weights)
    out = pl.pallas_call(
        _combine_body,
        out_shape=jax.ShapeDtypeStruct((3, E), jnp.float32),
    )(p_l, p_w, colsum)
    return out


# speedup vs baseline: 1.0683x; 1.0683x over previous
import functools

import jax
import jax.numpy as jnp
from jax import lax
from jax.experimental import pallas as pl
from jax.experimental.pallas import tpu as pltpu
from jax.experimental.pallas import tpu_sc as plsc

TOKENS = 16384
E = 64
K = 8
L = 16
NC = 2
NS = 16
NW = NC * NS
ROWS = TOKENS // NW
BLK = 512
GRID = TOKENS // BLK

_mesh = plsc.VectorSubcoreMesh(core_axis_name="c", subcore_axis_name="s")


@functools.partial(
    pl.kernel,
    out_type=jax.ShapeDtypeStruct((NW, 2 * E), jnp.float32),
    mesh=_mesh,
    compiler_params=pltpu.CompilerParams(needs_layout_passes=False),
    scratch_types=[
        pltpu.VMEM((ROWS // 2, E), jnp.float32),
        pltpu.VMEM((ROWS // 2, E), jnp.float32),
        pltpu.VMEM((2 * E,), jnp.float32),
        pltpu.SemaphoreType.DMA,
        pltpu.SemaphoreType.DMA,
    ],
)
def _sc_topk_hist(l_hbm, w_hbm, out_hbm, buf_a, buf_b, acc_v, sem_a, sem_b):
    c = lax.axis_index("c")
    s = lax.axis_index("s")
    wid = s * NC + c
    base = wid * ROWS

    iota = lax.iota(jnp.int32, L)
    zeros = jnp.zeros((L,), jnp.float32)
    ones = jnp.ones((L,), jnp.float32)
    low_half = iota < 8
    shift8 = (iota + 8) & 15
    idx_l = [iota + L * j for j in range(E // L)]
    idx_w = [iota + L * j + E for j in range(E // L)]
    for j in range(2 * E // L):
        acc_v[pl.ds(L * j, L)] = zeros

    def merge(ka, va, kb, vb, descending):
        take_a = ka >= kb
        mk = jnp.maximum(ka, kb)
        mv = jnp.where(take_a, va, vb)
        return plsc.sort_key_val(mk, mv, descending=descending)

    def top8(buf, r, consts):
        ks, vs = [], []
        for j in range(E // L):
            k_s, v_s = plsc.sort_key_val(
                buf[r, pl.ds(L * j, L)], consts[j],
                descending=(j % 2 == 0),
            )
            ks.append(k_s)
            vs.append(v_s)
        k01, v01 = merge(ks[0], vs[0], ks[1], vs[1], descending=True)
        k23, v23 = merge(ks[2], vs[2], ks[3], vs[3], descending=False)
        take_a = k01 >= k23
        kv = jnp.maximum(k01, k23)
        vv = jnp.where(take_a, v01, v23)
        kv_s = lax.gather(
            kv, shift8[:, None],
            dimension_numbers=lax.GatherDimensionNumbers(
                offset_dims=(), collapsed_slice_dims=(0,),
                start_index_map=(0,),
            ),
            slice_sizes=(1,),
            mode=lax.GatherScatterMode.PROMISE_IN_BOUNDS,
        )
        sel = jnp.where(low_half, kv >= kv_s, kv > kv_s)
        return vv, sel

    def loop_hist(buf, consts):
        @plsc.parallel_loop(0, ROWS // 2, unroll=4)
        def _(r):
            vv, sel = top8(buf, r, consts)
            plsc.addupdate_scatter(acc_v, [vv], ones, mask=sel)

    CH = ROWS // 2
    h_a = pltpu.async_copy(l_hbm.at[pl.ds(base, CH)], buf_a, sem_a)
    h_b = pltpu.async_copy(l_hbm.at[pl.ds(base + CH, CH)], buf_b, sem_b)
    h_a.wait()
    loop_hist(buf_a, idx_l)
    h_a2 = pltpu.async_copy(w_hbm.at[pl.ds(base, CH)], buf_a, sem_a)
    h_b.wait()
    loop_hist(buf_b, idx_l)
    h_b2 = pltpu.async_copy(w_hbm.at[pl.ds(base + CH, CH)], buf_b, sem_b)
    h_a2.wait()
    loop_hist(buf_a, idx_w)
    h_b2.wait()
    loop_hist(buf_b, idx_w)

    pltpu.sync_copy(acc_v, out_hbm.at[wid])


def _tc_colsum_body(x_ref, o_ref):
    o_ref[...] = jnp.sum(x_ref[...], axis=0, keepdims=True)


def _combine_body(p_ref, cs_ref, o_ref):
    s = jnp.sum(p_ref[...], axis=0, keepdims=True)
    o_ref[0:1, :] = s[:, 0:E]
    o_ref[1:2, :] = s[:, E:2 * E]
    o_ref[2:3, :] = cs_ref[...]


def kernel(layer_idx, router_weights, num_experts_per_tok, router_logits):
    partials = _sc_topk_hist(router_logits, router_weights)
    colsum = pl.pallas_call(
        _tc_colsum_body,
        out_shape=jax.ShapeDtypeStruct((1, E), jnp.float32),
    )(router_weights)
    out = pl.pallas_call(
        _combine_body,
        out_shape=jax.ShapeDtypeStruct((3, E), jnp.float32),
    )(partials, colsum)
    return out
